# baseline (speedup 1.0000x reference)
import functools
import jax
import jax.numpy as jnp
from jax import lax
from jax.experimental import pallas as pl
from jax.experimental.pallas import tpu as pltpu
from jax.experimental.pallas import tpu_sc as plsc

_BN = 32768
_RB = 256
_TC_UNITS = 4
_NW = 32
_CH = 256


def _matvec_kernel(x_ref, w_ref, b_ref, out_ref, k_ref, acc):
    i = pl.program_id(0)
    wsum = jnp.sum(w_ref[...], axis=0, keepdims=True)
    bsum = jnp.sum(b_ref[...])
    col = jax.lax.dot_general(
        x_ref[...], wsum,
        dimension_numbers=(((1,), (1,)), ((), ())),
        preferred_element_type=jnp.float32)
    s = col.reshape(_BN // 128, 128) + bsum
    out_ref[...] = s

    @pl.when(i == 0)
    def _init():
        acc[0] = 0
    acc[0] = acc[0] + jnp.sum((s != 0.0).astype(jnp.float32)).astype(jnp.int32)
    k_ref[0, 0] = acc[0]


def _sc_body(nchunks, tc_rows, x_hbm, w_hbm, b_hbm, out_hbm, cnt_hbm,
             w_v, b_v, buf0, buf1, out_v, cnt_v, wsum_v, sem0, sem1):
    wid = lax.axis_index("s") * 2 + lax.axis_index("c")
    rpw = nchunks * _CH
    base = tc_rows + wid * rpw

    pltpu.sync_copy(w_hbm, w_v)
    pltpu.sync_copy(b_hbm, b_v)
    for cc in range(8):
        accw = jnp.zeros((16,), jnp.float32)
        for i in range(16):
            accw = accw + w_v[pl.ds(i * 128 + cc * 16, 16)]
        wsum_v[pl.ds(cc * 16, 16)] = accw
    bsum = jnp.sum(b_v[...])

    row_iota = lax.iota(jnp.int32, 16)

    def compute_chunk(c, buf, cnt_acc):
        def group(g, cnt_in):
            rows = g * 16 + row_iota

            def col(dd, acc):
                idx = jnp.zeros((16,), jnp.int32) + dd
                wv = plsc.load_gather(wsum_v, [idx])
                v = plsc.load_gather(buf, [rows, idx])
                return acc + v * wv
            s = lax.fori_loop(0, 128, col, jnp.zeros((16,), jnp.float32))
            s = s + bsum
            out_v[pl.ds(c * _CH + g * 16, 16)] = s
            return cnt_in + jnp.where(s != 0.0, 1.0, 0.0)
        return lax.fori_loop(0, _CH // 16, group, cnt_acc)

    bufs = (buf0, buf1)
    sems = (sem0, sem1)
    cnt_acc = jnp.zeros((16,), jnp.float32)
    pend = pltpu.async_copy(x_hbm.at[pl.ds(base, _CH), :], buf0, sem0)
    for c in range(nchunks):
        nxt = None
        if c + 1 < nchunks:
            nxt = pltpu.async_copy(
                x_hbm.at[pl.ds(base + (c + 1) * _CH, _CH), :],
                bufs[(c + 1) % 2], sems[(c + 1) % 2])
        pend.wait()
        cnt_acc = compute_chunk(c, bufs[c % 2], cnt_acc)
        pend = nxt
    cnt_v[...] = cnt_acc
    pltpu.sync_copy(out_v, out_hbm.at[pl.ds(wid * rpw, rpw)])
    pltpu.sync_copy(cnt_v, cnt_hbm.at[pl.ds(wid * 16, 16)])


def _sc_matvec(x, W, b, tc_rows):
    N = x.shape[0]
    M = N - tc_rows
    rpw = M // _NW
    nchunks = rpw // _CH
    mesh = plsc.VectorSubcoreMesh(core_axis_name="c", subcore_axis_name="s",
                                  num_cores=2, num_subcores=16)
    kfn = functools.partial(
        pl.kernel,
        out_type=[
            jax.ShapeDtypeStruct((M,), jnp.float32),
            jax.ShapeDtypeStruct((512,), jnp.float32),
        ],
        mesh=mesh,
        scratch_types=[
            pltpu.VMEM((2048,), jnp.float32),
            pltpu.VMEM((16,), jnp.float32),
            pltpu.VMEM((_CH, 128), jnp.float32),
            pltpu.VMEM((_CH, 128), jnp.float32),
            pltpu.VMEM((rpw,), jnp.float32),
            pltpu.VMEM((16,), jnp.float32),
            pltpu.VMEM((128,), jnp.float32),
            pltpu.SemaphoreType.DMA,
            pltpu.SemaphoreType.DMA,
        ],
        compiler_params=pltpu.CompilerParams(needs_layout_passes=False),
    )(functools.partial(_sc_body, nchunks, tc_rows))
    return kfn(x, W.reshape(-1), b)


def _mask_kernel(s_ref, k_ref, c_ref, o_ref, sm):
    j = pl.program_id(0)
    s = s_ref[...]
    nz = (s != 0.0)
    mi = nz.astype(jnp.float32)

    @pl.when(j == 0)
    def _init():
        sm[0] = 0

    k = k_ref[0, 0] + jnp.sum(c_ref[...]).astype(jnp.int32)
    start = jnp.maximum(k // 2, 1)
    d = jax.lax.broadcasted_iota(jnp.int32, (128, 128), 0)
    l = jax.lax.broadcasted_iota(jnp.int32, (128, 128), 1)
    tri = (d <= l).astype(jnp.float32)
    incl = jax.lax.dot(mi, tri,
                       preferred_element_type=jnp.float32)
    sel = (d == 127).astype(jnp.float32)
    rowcnt = jax.lax.dot(incl, sel,
                         preferred_element_type=jnp.float32)
    r2 = jax.lax.broadcasted_iota(jnp.int32, (_RB, _RB), 0)
    q2 = jax.lax.broadcasted_iota(jnp.int32, (_RB, _RB), 1)
    low = (q2 < r2).astype(jnp.float32)
    rowoff = jax.lax.dot(low, rowcnt,
                         preferred_element_type=jnp.float32)
    carry = sm[0].astype(jnp.float32)
    rank = carry + rowoff + (incl - mi)
    keep = nz & (rank >= start.astype(jnp.float32))
    o_ref[...] = jnp.where(keep, s, 0.0)
    sm[0] = sm[0] + jnp.sum(mi).astype(jnp.int32)


def kernel(x, W, b):
    N, D = x.shape
    R = N // 128
    tc_rows = _TC_UNITS * _BN
    r_tc = tc_rows // 128
    b2d = b.reshape(1, b.shape[0])

    sums_tc, kval = pl.pallas_call(
        _matvec_kernel,
        grid=(tc_rows // _BN,),
        in_specs=[
            pl.BlockSpec((_BN, D), lambda i: (i, 0)),
            pl.BlockSpec((W.shape[0], D), lambda i: (0, 0)),
            pl.BlockSpec((1, b.shape[0]), lambda i: (0, 0)),
        ],
        out_specs=[
            pl.BlockSpec((_BN // 128, 128), lambda i: (i, 0)),
            pl.BlockSpec(memory_space=pltpu.SMEM),
        ],
        out_shape=[
            jax.ShapeDtypeStruct((r_tc, 128), jnp.float32),
            jax.ShapeDtypeStruct((1, 1), jnp.int32),
        ],
        scratch_shapes=[pltpu.SMEM((1,), jnp.int32)],
        compiler_params=pltpu.CompilerParams(
            dimension_semantics=("arbitrary",)),
    )(x, W, b2d)

    sums_sc, cnt_sc = _sc_matvec(x, W, b, tc_rows)

    sums2d = jnp.concatenate(
        [sums_tc, sums_sc.reshape(-1, 128)], axis=0)
    cnt2d = cnt_sc.reshape(4, 128)

    out2d = pl.pallas_call(
        _mask_kernel,
        grid=(R // _RB,),
        in_specs=[
            pl.BlockSpec((_RB, 128), lambda j: (j, 0)),
            pl.BlockSpec(memory_space=pltpu.SMEM),
            pl.BlockSpec((4, 128), lambda j: (0, 0)),
        ],
        out_specs=pl.BlockSpec((_RB, 128), lambda j: (j, 0)),
        out_shape=jax.ShapeDtypeStruct((R, 128), jnp.float32),
        scratch_shapes=[pltpu.SMEM((1,), jnp.int32)],
        compiler_params=pltpu.CompilerParams(
            dimension_semantics=("arbitrary",)),
    )(sums2d, kval, cnt2d)
    return out2d.reshape(N)


# speedup vs baseline: 5.2295x; 5.2295x over previous
import functools
import jax
import jax.numpy as jnp
from jax import lax
from jax.experimental import pallas as pl
from jax.experimental.pallas import tpu as pltpu
from jax.experimental.pallas import tpu_sc as plsc

_BN = 32768
_RB = 256
_TC_UNITS = 4
_NW = 32
_CH = 256


def _matvec_kernel(x_ref, w_ref, b_ref, out_ref, k_ref, acc):
    i = pl.program_id(0)
    wsum = jnp.sum(w_ref[...], axis=0, keepdims=True)
    bsum = jnp.sum(b_ref[...])
    col = jax.lax.dot_general(
        x_ref[...], wsum,
        dimension_numbers=(((1,), (1,)), ((), ())),
        preferred_element_type=jnp.float32)
    s = col.reshape(_BN // 128, 128) + bsum
    out_ref[...] = s

    @pl.when(i == 0)
    def _init():
        acc[0] = 0
    acc[0] = acc[0] + jnp.sum((s != 0.0).astype(jnp.float32)).astype(jnp.int32)
    k_ref[0, 0] = acc[0]


def _sc_body(nchunks, tc_rows, x_hbm, w_hbm, b_hbm, out_hbm, cnt_hbm,
             w_v, b_v, buf0, buf1, out_v, cnt_v, wsum_v, tr_v, sem0, sem1):
    wid = lax.axis_index("s") * 2 + lax.axis_index("c")
    rpw = nchunks * _CH
    base = tc_rows + wid * rpw

    pltpu.sync_copy(w_hbm, w_v)
    pltpu.sync_copy(b_hbm, b_v)
    for cc in range(8):
        accw = jnp.zeros((16,), jnp.float32)
        for i in range(16):
            accw = accw + w_v[pl.ds(i * 128 + cc * 16, 16)]
        wsum_v[pl.ds(cc * 16, 16)] = accw
    bsum = jnp.sum(b_v[...])
    ws = [wsum_v[pl.ds(cc * 16, 16)] for cc in range(8)]

    row_iota = lax.iota(jnp.int32, 16)

    def compute_chunk(c, buf, cnt_acc):
        def group(g, cnt_in):
            accs = []
            for r in range(16):
                acc = buf[g * 16 + r, pl.ds(0, 16)] * ws[0]
                for cc in range(1, 8):
                    acc = acc + buf[g * 16 + r, pl.ds(cc * 16, 16)] * ws[cc]
                accs.append(acc)
            for r in range(16):
                tr_v[pl.ds(r * 16, 16)] = accs[r]
            s = jnp.zeros((16,), jnp.float32) + bsum
            for cc in range(16):
                s = s + plsc.load_gather(tr_v, [row_iota * 16 + cc])
            out_v[pl.ds(c * _CH + g * 16, 16)] = s
            return cnt_in + jnp.where(s != 0.0, 1.0, 0.0)
        return lax.fori_loop(0, _CH // 16, group, cnt_acc)

    def wait_buf(buf, sem):
        pltpu.make_async_copy(
            x_hbm.at[pl.ds(base, _CH), :], buf, sem).wait()

    def start_chunk(c, buf, sem):
        pltpu.async_copy(x_hbm.at[pl.ds(base + c * _CH, _CH), :], buf, sem)

    start_chunk(0, buf0, sem0)
    start_chunk(1, buf1, sem1)

    def pair(i, cnt_acc):
        c0 = i * 2
        wait_buf(buf0, sem0)
        cnt_acc = compute_chunk(c0, buf0, cnt_acc)

        @pl.when(c0 + 2 < nchunks)
        def _():
            start_chunk(c0 + 2, buf0, sem0)
        wait_buf(buf1, sem1)
        cnt_acc = compute_chunk(c0 + 1, buf1, cnt_acc)

        @pl.when(c0 + 3 < nchunks)
        def _():
            start_chunk(c0 + 3, buf1, sem1)
        return cnt_acc

    cnt_acc = lax.fori_loop(0, nchunks // 2, pair,
                            jnp.zeros((16,), jnp.float32))
    cnt_v[...] = cnt_acc
    pltpu.sync_copy(out_v, out_hbm.at[pl.ds(wid * rpw, rpw)])
    pltpu.sync_copy(cnt_v, cnt_hbm.at[pl.ds(wid * 16, 16)])


def _sc_matvec(x, W, b, tc_rows):
    N = x.shape[0]
    M = N - tc_rows
    rpw = M // _NW
    nchunks = rpw // _CH
    mesh = plsc.VectorSubcoreMesh(core_axis_name="c", subcore_axis_name="s",
                                  num_cores=2, num_subcores=16)
    kfn = functools.partial(
        pl.kernel,
        out_type=[
            jax.ShapeDtypeStruct((M,), jnp.float32),
            jax.ShapeDtypeStruct((512,), jnp.float32),
        ],
        mesh=mesh,
        scratch_types=[
            pltpu.VMEM((2048,), jnp.float32),
            pltpu.VMEM((16,), jnp.float32),
            pltpu.VMEM((_CH, 128), jnp.float32),
            pltpu.VMEM((_CH, 128), jnp.float32),
            pltpu.VMEM((rpw,), jnp.float32),
            pltpu.VMEM((16,), jnp.float32),
            pltpu.VMEM((128,), jnp.float32),
            pltpu.VMEM((256,), jnp.float32),
            pltpu.SemaphoreType.DMA,
            pltpu.SemaphoreType.DMA,
        ],
        compiler_params=pltpu.CompilerParams(needs_layout_passes=False),
    )(functools.partial(_sc_body, nchunks, tc_rows))
    return kfn(x, W.reshape(-1), b)


def _mask_kernel(s_ref, k_ref, c_ref, o_ref, sm):
    j = pl.program_id(0)
    s = s_ref[...]
    nz = (s != 0.0)
    mi = nz.astype(jnp.float32)

    @pl.when(j == 0)
    def _init():
        sm[0] = 0

    k = k_ref[0, 0] + jnp.sum(c_ref[...]).astype(jnp.int32)
    start = jnp.maximum(k // 2, 1)
    d = jax.lax.broadcasted_iota(jnp.int32, (128, 128), 0)
    l = jax.lax.broadcasted_iota(jnp.int32, (128, 128), 1)
    tri = (d <= l).astype(jnp.float32)
    incl = jax.lax.dot(mi, tri,
                       preferred_element_type=jnp.float32)
    sel = (d == 127).astype(jnp.float32)
    rowcnt = jax.lax.dot(incl, sel,
                         preferred_element_type=jnp.float32)
    r2 = jax.lax.broadcasted_iota(jnp.int32, (_RB, _RB), 0)
    q2 = jax.lax.broadcasted_iota(jnp.int32, (_RB, _RB), 1)
    low = (q2 < r2).astype(jnp.float32)
    rowoff = jax.lax.dot(low, rowcnt,
                         preferred_element_type=jnp.float32)
    carry = sm[0].astype(jnp.float32)
    rank = carry + rowoff + (incl - mi)
    keep = nz & (rank >= start.astype(jnp.float32))
    o_ref[...] = jnp.where(keep, s, 0.0)
    sm[0] = sm[0] + jnp.sum(mi).astype(jnp.int32)


def kernel(x, W, b):
    N, D = x.shape
    R = N // 128
    tc_rows = _TC_UNITS * _BN
    r_tc = tc_rows // 128
    b2d = b.reshape(1, b.shape[0])

    sums_tc, kval = pl.pallas_call(
        _matvec_kernel,
        grid=(tc_rows // _BN,),
        in_specs=[
            pl.BlockSpec((_BN, D), lambda i: (i, 0)),
            pl.BlockSpec((W.shape[0], D), lambda i: (0, 0)),
            pl.BlockSpec((1, b.shape[0]), lambda i: (0, 0)),
        ],
        out_specs=[
            pl.BlockSpec((_BN // 128, 128), lambda i: (i, 0)),
            pl.BlockSpec(memory_space=pltpu.SMEM),
        ],
        out_shape=[
            jax.ShapeDtypeStruct((r_tc, 128), jnp.float32),
            jax.ShapeDtypeStruct((1, 1), jnp.int32),
        ],
        scratch_shapes=[pltpu.SMEM((1,), jnp.int32)],
        compiler_params=pltpu.CompilerParams(
            dimension_semantics=("arbitrary",)),
    )(x, W, b2d)

    sums_sc, cnt_sc = _sc_matvec(x, W, b, tc_rows)

    sums2d = jnp.concatenate(
        [sums_tc, sums_sc.reshape(-1, 128)], axis=0)
    cnt2d = cnt_sc.reshape(4, 128)

    out2d = pl.pallas_call(
        _mask_kernel,
        grid=(R // _RB,),
        in_specs=[
            pl.BlockSpec((_RB, 128), lambda j: (j, 0)),
            pl.BlockSpec(memory_space=pltpu.SMEM),
            pl.BlockSpec((4, 128), lambda j: (0, 0)),
        ],
        out_specs=pl.BlockSpec((_RB, 128), lambda j: (j, 0)),
        out_shape=jax.ShapeDtypeStruct((R, 128), jnp.float32),
        scratch_shapes=[pltpu.SMEM((1,), jnp.int32)],
        compiler_params=pltpu.CompilerParams(
            dimension_semantics=("arbitrary",)),
    )(sums2d, kval, cnt2d)
    return out2d.reshape(N)


# speedup vs baseline: 5.2551x; 1.0049x over previous
import functools
import jax
import jax.numpy as jnp
from jax import lax
from jax.experimental import pallas as pl
from jax.experimental.pallas import tpu as pltpu
from jax.experimental.pallas import tpu_sc as plsc

_BN = 32768
_RB = 256
_TC_UNITS = 4
_NW = 32
_CH = 256


def _matvec_kernel(x_ref, w_ref, b_ref, out_ref, k_ref, acc):
    i = pl.program_id(0)
    wsum = jnp.sum(w_ref[...], axis=0, keepdims=True)
    bsum = jnp.sum(b_ref[...])
    col = jax.lax.dot_general(
        x_ref[...], wsum,
        dimension_numbers=(((1,), (1,)), ((), ())),
        preferred_element_type=jnp.float32)
    s = col.reshape(_BN // 128, 128) + bsum
    out_ref[...] = s

    @pl.when(i == 0)
    def _init():
        acc[0] = 0
    acc[0] = acc[0] + jnp.sum((s != 0.0).astype(jnp.float32)).astype(jnp.int32)
    k_ref[0, 0] = acc[0]


def _sc_body(nchunks, tc_rows, x_hbm, w_hbm, b_hbm, out_hbm, cnt_hbm,
             w_v, b_v, buf0, buf1, out_v, cnt_v, wsum_v, tr_v, sem0, sem1):
    wid = lax.axis_index("s") * 2 + lax.axis_index("c")
    rpw = nchunks * _CH
    base = tc_rows + wid * rpw

    pltpu.sync_copy(w_hbm, w_v)
    pltpu.sync_copy(b_hbm, b_v)
    for cc in range(8):
        accw = jnp.zeros((16,), jnp.float32)
        for i in range(16):
            accw = accw + w_v[pl.ds(i * 128 + cc * 16, 16)]
        wsum_v[pl.ds(cc * 16, 16)] = accw
    bsum = jnp.sum(b_v[...])
    ws = [wsum_v[pl.ds(cc * 16, 16)] for cc in range(8)]

    row_iota = lax.iota(jnp.int32, 16)

    def compute_chunk(c, buf, cnt_acc):
        def group(g, cnt_in):
            accs = []
            for r in range(16):
                acc = buf[g * 16 + r, pl.ds(0, 16)] * ws[0]
                for cc in range(1, 8):
                    acc = acc + buf[g * 16 + r, pl.ds(cc * 16, 16)] * ws[cc]
                accs.append(acc)
            for r in range(16):
                tr_v[pl.ds(r * 16, 16)] = accs[r]
            s = jnp.zeros((16,), jnp.float32) + bsum
            for cc in range(16):
                s = s + plsc.load_gather(tr_v, [row_iota * 16 + cc])
            out_v[pl.ds(c * _CH + g * 16, 16)] = s
            return cnt_in + jnp.where(s != 0.0, 1.0, 0.0)
        return lax.fori_loop(0, _CH // 16, group, cnt_acc)

    def wait_buf(buf, sem):
        pltpu.make_async_copy(
            x_hbm.at[pl.ds(base, _CH), :], buf, sem).wait()

    def start_chunk(c, buf, sem):
        pltpu.async_copy(x_hbm.at[pl.ds(base + c * _CH, _CH), :], buf, sem)

    start_chunk(0, buf0, sem0)
    start_chunk(1, buf1, sem1)

    def pair(i, cnt_acc):
        c0 = i * 2
        wait_buf(buf0, sem0)
        cnt_acc = compute_chunk(c0, buf0, cnt_acc)

        @pl.when(c0 + 2 < nchunks)
        def _():
            start_chunk(c0 + 2, buf0, sem0)
        wait_buf(buf1, sem1)
        cnt_acc = compute_chunk(c0 + 1, buf1, cnt_acc)

        @pl.when(c0 + 3 < nchunks)
        def _():
            start_chunk(c0 + 3, buf1, sem1)
        return cnt_acc

    cnt_acc = lax.fori_loop(0, nchunks // 2, pair,
                            jnp.zeros((16,), jnp.float32))
    cnt_v[...] = cnt_acc
    pltpu.sync_copy(out_v, out_hbm.at[pl.ds(wid * rpw, rpw)])
    pltpu.sync_copy(cnt_v, cnt_hbm.at[pl.ds(wid * 16, 16)])


def _sc_matvec(x, W, b, tc_rows):
    N = x.shape[0]
    M = N - tc_rows
    rpw = M // _NW
    nchunks = rpw // _CH
    mesh = plsc.VectorSubcoreMesh(core_axis_name="c", subcore_axis_name="s",
                                  num_cores=2, num_subcores=16)
    kfn = functools.partial(
        pl.kernel,
        out_type=[
            jax.ShapeDtypeStruct((M,), jnp.float32),
            jax.ShapeDtypeStruct((512,), jnp.float32),
        ],
        mesh=mesh,
        scratch_types=[
            pltpu.VMEM((2048,), jnp.float32),
            pltpu.VMEM((16,), jnp.float32),
            pltpu.VMEM((_CH, 128), jnp.float32),
            pltpu.VMEM((_CH, 128), jnp.float32),
            pltpu.VMEM((rpw,), jnp.float32),
            pltpu.VMEM((16,), jnp.float32),
            pltpu.VMEM((128,), jnp.float32),
            pltpu.VMEM((256,), jnp.float32),
            pltpu.SemaphoreType.DMA,
            pltpu.SemaphoreType.DMA,
        ],
        compiler_params=pltpu.CompilerParams(needs_layout_passes=False),
    )(functools.partial(_sc_body, nchunks, tc_rows))
    return kfn(x, W.reshape(-1), b)


def _mask_kernel(s_ref, k_ref, c_ref, o_ref, sm):
    j = pl.program_id(0)
    s = s_ref[...]
    nz = (s != 0.0)
    mi = nz.astype(jnp.float32)

    @pl.when(j == 0)
    def _init():
        sm[0] = 0

    k = k_ref[0, 0] + jnp.sum(c_ref[...]).astype(jnp.int32)
    start = jnp.maximum(k // 2, 1)
    d = jax.lax.broadcasted_iota(jnp.int32, (128, 128), 0)
    l = jax.lax.broadcasted_iota(jnp.int32, (128, 128), 1)
    tri = (d <= l).astype(jnp.float32)
    incl = jax.lax.dot(mi, tri,
                       preferred_element_type=jnp.float32)
    sel = (d == 127).astype(jnp.float32)
    rowcnt = jax.lax.dot(incl, sel,
                         preferred_element_type=jnp.float32)
    r2 = jax.lax.broadcasted_iota(jnp.int32, (_RB, _RB), 0)
    q2 = jax.lax.broadcasted_iota(jnp.int32, (_RB, _RB), 1)
    low = (q2 < r2).astype(jnp.float32)
    rowoff = jax.lax.dot(low, rowcnt,
                         preferred_element_type=jnp.float32)
    carry = sm[0].astype(jnp.float32)
    rank = carry + rowoff + (incl - mi)
    keep = nz & (rank >= start.astype(jnp.float32))
    o_ref[...] = jnp.where(keep, s, 0.0)
    sm[0] = sm[0] + jnp.sum(mi).astype(jnp.int32)


def kernel(x, W, b):
    N, D = x.shape
    R = N // 128
    tc_rows = _TC_UNITS * _BN
    r_tc = tc_rows // 128
    b2d = b.reshape(1, b.shape[0])

    sums_sc, cnt_sc = _sc_matvec(x, W, b, tc_rows)

    sums_tc, kval = pl.pallas_call(
        _matvec_kernel,
        grid=(tc_rows // _BN,),
        in_specs=[
            pl.BlockSpec((_BN, D), lambda i: (i, 0)),
            pl.BlockSpec((W.shape[0], D), lambda i: (0, 0)),
            pl.BlockSpec((1, b.shape[0]), lambda i: (0, 0)),
        ],
        out_specs=[
            pl.BlockSpec((_BN // 128, 128), lambda i: (i, 0)),
            pl.BlockSpec(memory_space=pltpu.SMEM),
        ],
        out_shape=[
            jax.ShapeDtypeStruct((r_tc, 128), jnp.float32),
            jax.ShapeDtypeStruct((1, 1), jnp.int32),
        ],
        scratch_shapes=[pltpu.SMEM((1,), jnp.int32)],
        compiler_params=pltpu.CompilerParams(
            dimension_semantics=("arbitrary",)),
    )(x, W, b2d)

    sums2d = jnp.concatenate(
        [sums_tc, sums_sc.reshape(-1, 128)], axis=0)
    cnt2d = cnt_sc.reshape(4, 128)

    out2d = pl.pallas_call(
        _mask_kernel,
        grid=(R // _RB,),
        in_specs=[
            pl.BlockSpec((_RB, 128), lambda j: (j, 0)),
            pl.BlockSpec(memory_space=pltpu.SMEM),
            pl.BlockSpec((4, 128), lambda j: (0, 0)),
        ],
        out_specs=pl.BlockSpec((_RB, 128), lambda j: (j, 0)),
        out_shape=jax.ShapeDtypeStruct((R, 128), jnp.float32),
        scratch_shapes=[pltpu.SMEM((1,), jnp.int32)],
        compiler_params=pltpu.CompilerParams(
            dimension_semantics=("arbitrary",)),
    )(sums2d, kval, cnt2d)
    return out2d.reshape(N)


# speedup vs baseline: 5.7106x; 1.0867x over previous
import functools
import jax
import jax.numpy as jnp
from jax import lax
from jax.experimental import pallas as pl
from jax.experimental.pallas import tpu as pltpu
from jax.experimental.pallas import tpu_sc as plsc

_BN = 32768
_RB = 256
_TC_UNITS = 6
_NW = 32
_CH = 256


def _matvec_kernel(x_ref, w_ref, b_ref, out_ref, k_ref, acc):
    i = pl.program_id(0)
    wsum = jnp.sum(w_ref[...], axis=0, keepdims=True)
    bsum = jnp.sum(b_ref[...])
    col = jax.lax.dot_general(
        x_ref[...], wsum,
        dimension_numbers=(((1,), (1,)), ((), ())),
        preferred_element_type=jnp.float32)
    s = col.reshape(_BN // 128, 128) + bsum
    out_ref[...] = s

    @pl.when(i == 0)
    def _init():
        acc[0] = 0
    acc[0] = acc[0] + jnp.sum((s != 0.0).astype(jnp.float32)).astype(jnp.int32)
    k_ref[0, 0] = acc[0]


def _sc_body(nchunks, tc_rows, x_hbm, w_hbm, b_hbm, out_hbm, cnt_hbm,
             w_v, b_v, buf0, buf1, out_v, cnt_v, wsum_v, tr_v, sem0, sem1):
    wid = lax.axis_index("s") * 2 + lax.axis_index("c")
    rpw = nchunks * _CH
    base = tc_rows + wid * rpw

    pltpu.sync_copy(w_hbm, w_v)
    pltpu.sync_copy(b_hbm, b_v)
    for cc in range(8):
        accw = jnp.zeros((16,), jnp.float32)
        for i in range(16):
            accw = accw + w_v[pl.ds(i * 128 + cc * 16, 16)]
        wsum_v[pl.ds(cc * 16, 16)] = accw
    bsum = jnp.sum(b_v[...])
    ws = [wsum_v[pl.ds(cc * 16, 16)] for cc in range(8)]

    row_iota = lax.iota(jnp.int32, 16)

    def compute_chunk(c, buf, cnt_acc):
        def group(g, cnt_in):
            accs = []
            for r in range(16):
                acc = buf[g * 16 + r, pl.ds(0, 16)] * ws[0]
                for cc in range(1, 8):
                    acc = acc + buf[g * 16 + r, pl.ds(cc * 16, 16)] * ws[cc]
                accs.append(acc)
            for r in range(16):
                tr_v[pl.ds(r * 16, 16)] = accs[r]
            s = jnp.zeros((16,), jnp.float32) + bsum
            for cc in range(16):
                s = s + plsc.load_gather(tr_v, [row_iota * 16 + cc])
            out_v[pl.ds(c * _CH + g * 16, 16)] = s
            return cnt_in + jnp.where(s != 0.0, 1.0, 0.0)
        return lax.fori_loop(0, _CH // 16, group, cnt_acc)

    def wait_buf(buf, sem):
        pltpu.make_async_copy(
            x_hbm.at[pl.ds(base, _CH), :], buf, sem).wait()

    def start_chunk(c, buf, sem):
        pltpu.async_copy(x_hbm.at[pl.ds(base + c * _CH, _CH), :], buf, sem)

    start_chunk(0, buf0, sem0)
    start_chunk(1, buf1, sem1)

    def pair(i, cnt_acc):
        c0 = i * 2
        wait_buf(buf0, sem0)
        cnt_acc = compute_chunk(c0, buf0, cnt_acc)

        @pl.when(c0 + 2 < nchunks)
        def _():
            start_chunk(c0 + 2, buf0, sem0)
        wait_buf(buf1, sem1)
        cnt_acc = compute_chunk(c0 + 1, buf1, cnt_acc)

        @pl.when(c0 + 3 < nchunks)
        def _():
            start_chunk(c0 + 3, buf1, sem1)
        return cnt_acc

    cnt_acc = lax.fori_loop(0, nchunks // 2, pair,
                            jnp.zeros((16,), jnp.float32))
    cnt_v[...] = cnt_acc
    pltpu.sync_copy(out_v, out_hbm.at[pl.ds(wid * rpw, rpw)])
    pltpu.sync_copy(cnt_v, cnt_hbm.at[pl.ds(wid * 16, 16)])


def _sc_matvec(x, W, b, tc_rows):
    N = x.shape[0]
    M = N - tc_rows
    rpw = M // _NW
    nchunks = rpw // _CH
    mesh = plsc.VectorSubcoreMesh(core_axis_name="c", subcore_axis_name="s",
                                  num_cores=2, num_subcores=16)
    kfn = functools.partial(
        pl.kernel,
        out_type=[
            jax.ShapeDtypeStruct((M,), jnp.float32),
            jax.ShapeDtypeStruct((512,), jnp.float32),
        ],
        mesh=mesh,
        scratch_types=[
            pltpu.VMEM((2048,), jnp.float32),
            pltpu.VMEM((16,), jnp.float32),
            pltpu.VMEM((_CH, 128), jnp.float32),
            pltpu.VMEM((_CH, 128), jnp.float32),
            pltpu.VMEM((rpw,), jnp.float32),
            pltpu.VMEM((16,), jnp.float32),
            pltpu.VMEM((128,), jnp.float32),
            pltpu.VMEM((256,), jnp.float32),
            pltpu.SemaphoreType.DMA,
            pltpu.SemaphoreType.DMA,
        ],
        compiler_params=pltpu.CompilerParams(needs_layout_passes=False),
    )(functools.partial(_sc_body, nchunks, tc_rows))
    return kfn(x, W.reshape(-1), b)


def _mask_kernel(s_ref, k_ref, c_ref, o_ref, sm):
    j = pl.program_id(0)
    s = s_ref[...]
    nz = (s != 0.0)
    mi = nz.astype(jnp.float32)

    @pl.when(j == 0)
    def _init():
        sm[0] = 0

    k = k_ref[0, 0] + jnp.sum(c_ref[...]).astype(jnp.int32)
    start = jnp.maximum(k // 2, 1)
    d = jax.lax.broadcasted_iota(jnp.int32, (128, 128), 0)
    l = jax.lax.broadcasted_iota(jnp.int32, (128, 128), 1)
    tri = (d <= l).astype(jnp.float32)
    incl = jax.lax.dot(mi, tri,
                       preferred_element_type=jnp.float32)
    sel = (d == 127).astype(jnp.float32)
    rowcnt = jax.lax.dot(incl, sel,
                         preferred_element_type=jnp.float32)
    r2 = jax.lax.broadcasted_iota(jnp.int32, (_RB, _RB), 0)
    q2 = jax.lax.broadcasted_iota(jnp.int32, (_RB, _RB), 1)
    low = (q2 < r2).astype(jnp.float32)
    rowoff = jax.lax.dot(low, rowcnt,
                         preferred_element_type=jnp.float32)
    carry = sm[0].astype(jnp.float32)
    rank = carry + rowoff + (incl - mi)
    keep = nz & (rank >= start.astype(jnp.float32))
    o_ref[...] = jnp.where(keep, s, 0.0)
    sm[0] = sm[0] + jnp.sum(mi).astype(jnp.int32)


def kernel(x, W, b):
    N, D = x.shape
    R = N // 128
    tc_rows = _TC_UNITS * _BN
    r_tc = tc_rows // 128
    b2d = b.reshape(1, b.shape[0])

    sums_sc, cnt_sc = _sc_matvec(x, W, b, tc_rows)

    sums_tc, kval = pl.pallas_call(
        _matvec_kernel,
        grid=(tc_rows // _BN,),
        in_specs=[
            pl.BlockSpec((_BN, D), lambda i: (i, 0)),
            pl.BlockSpec((W.shape[0], D), lambda i: (0, 0)),
            pl.BlockSpec((1, b.shape[0]), lambda i: (0, 0)),
        ],
        out_specs=[
            pl.BlockSpec((_BN // 128, 128), lambda i: (i, 0)),
            pl.BlockSpec(memory_space=pltpu.SMEM),
        ],
        out_shape=[
            jax.ShapeDtypeStruct((r_tc, 128), jnp.float32),
            jax.ShapeDtypeStruct((1, 1), jnp.int32),
        ],
        scratch_shapes=[pltpu.SMEM((1,), jnp.int32)],
        compiler_params=pltpu.CompilerParams(
            dimension_semantics=("arbitrary",)),
    )(x, W, b2d)

    sums2d = jnp.concatenate(
        [sums_tc, sums_sc.reshape(-1, 128)], axis=0)
    cnt2d = cnt_sc.reshape(4, 128)

    out2d = pl.pallas_call(
        _mask_kernel,
        grid=(R // _RB,),
        in_specs=[
            pl.BlockSpec((_RB, 128), lambda j: (j, 0)),
            pl.BlockSpec(memory_space=pltpu.SMEM),
            pl.BlockSpec((4, 128), lambda j: (0, 0)),
        ],
        out_specs=pl.BlockSpec((_RB, 128), lambda j: (j, 0)),
        out_shape=jax.ShapeDtypeStruct((R, 128), jnp.float32),
        scratch_shapes=[pltpu.SMEM((1,), jnp.int32)],
        compiler_params=pltpu.CompilerParams(
            dimension_semantics=("arbitrary",)),
    )(sums2d, kval, cnt2d)
    return out2d.reshape(N)


# speedup vs baseline: 5.7604x; 1.0087x over previous
import functools
import jax
import jax.numpy as jnp
from jax import lax
from jax.experimental import pallas as pl
from jax.experimental.pallas import tpu as pltpu
from jax.experimental.pallas import tpu_sc as plsc

_BN = 32768
_RB = 256
_TC_UNITS = 5
_NW = 32
_CH = 256


def _matvec_kernel(x_ref, w_ref, b_ref, out_ref, k_ref, acc):
    i = pl.program_id(0)
    wsum = jnp.sum(w_ref[...], axis=0, keepdims=True)
    bsum = jnp.sum(b_ref[...])
    col = jax.lax.dot_general(
        x_ref[...], wsum,
        dimension_numbers=(((1,), (1,)), ((), ())),
        preferred_element_type=jnp.float32)
    s = col.reshape(_BN // 128, 128) + bsum
    out_ref[...] = s

    @pl.when(i == 0)
    def _init():
        acc[0] = 0
    acc[0] = acc[0] + jnp.sum((s != 0.0).astype(jnp.float32)).astype(jnp.int32)
    k_ref[0, 0] = acc[0]


def _sc_body(nchunks, tc_rows, x_hbm, w_hbm, b_hbm, out_hbm, cnt_hbm,
             w_v, b_v, buf0, buf1, out_v, cnt_v, wsum_v, tr_v, sem0, sem1):
    wid = lax.axis_index("s") * 2 + lax.axis_index("c")
    rpw = nchunks * _CH
    base = tc_rows + wid * rpw

    pltpu.sync_copy(w_hbm, w_v)
    pltpu.sync_copy(b_hbm, b_v)
    for cc in range(8):
        accw = jnp.zeros((16,), jnp.float32)
        for i in range(16):
            accw = accw + w_v[pl.ds(i * 128 + cc * 16, 16)]
        wsum_v[pl.ds(cc * 16, 16)] = accw
    bsum = jnp.sum(b_v[...])
    ws = [wsum_v[pl.ds(cc * 16, 16)] for cc in range(8)]

    row_iota = lax.iota(jnp.int32, 16)

    def compute_chunk(c, buf, cnt_acc):
        def group(g, cnt_in):
            accs = []
            for r in range(16):
                acc = buf[g * 16 + r, pl.ds(0, 16)] * ws[0]
                for cc in range(1, 8):
                    acc = acc + buf[g * 16 + r, pl.ds(cc * 16, 16)] * ws[cc]
                accs.append(acc)
            for r in range(16):
                tr_v[pl.ds(r * 16, 16)] = accs[r]
            s = jnp.zeros((16,), jnp.float32) + bsum
            for cc in range(16):
                s = s + plsc.load_gather(tr_v, [row_iota * 16 + cc])
            out_v[pl.ds(c * _CH + g * 16, 16)] = s
            return cnt_in + jnp.where(s != 0.0, 1.0, 0.0)
        return lax.fori_loop(0, _CH // 16, group, cnt_acc)

    def wait_buf(buf, sem):
        pltpu.make_async_copy(
            x_hbm.at[pl.ds(base, _CH), :], buf, sem).wait()

    def start_chunk(c, buf, sem):
        pltpu.async_copy(x_hbm.at[pl.ds(base + c * _CH, _CH), :], buf, sem)

    start_chunk(0, buf0, sem0)
    start_chunk(1, buf1, sem1)

    def pair(i, cnt_acc):
        c0 = i * 2
        wait_buf(buf0, sem0)
        cnt_acc = compute_chunk(c0, buf0, cnt_acc)

        @pl.when(c0 + 2 < nchunks)
        def _():
            start_chunk(c0 + 2, buf0, sem0)
        wait_buf(buf1, sem1)
        cnt_acc = compute_chunk(c0 + 1, buf1, cnt_acc)

        @pl.when(c0 + 3 < nchunks)
        def _():
            start_chunk(c0 + 3, buf1, sem1)
        return cnt_acc

    cnt_acc = lax.fori_loop(0, nchunks // 2, pair,
                            jnp.zeros((16,), jnp.float32))
    cnt_v[...] = cnt_acc
    pltpu.sync_copy(out_v, out_hbm.at[pl.ds(wid * rpw, rpw)])
    pltpu.sync_copy(cnt_v, cnt_hbm.at[pl.ds(wid * 16, 16)])


def _sc_matvec(x, W, b, tc_rows):
    N = x.shape[0]
    M = N - tc_rows
    rpw = M // _NW
    nchunks = rpw // _CH
    mesh = plsc.VectorSubcoreMesh(core_axis_name="c", subcore_axis_name="s",
                                  num_cores=2, num_subcores=16)
    kfn = functools.partial(
        pl.kernel,
        out_type=[
            jax.ShapeDtypeStruct((M,), jnp.float32),
            jax.ShapeDtypeStruct((512,), jnp.float32),
        ],
        mesh=mesh,
        scratch_types=[
            pltpu.VMEM((2048,), jnp.float32),
            pltpu.VMEM((16,), jnp.float32),
            pltpu.VMEM((_CH, 128), jnp.float32),
            pltpu.VMEM((_CH, 128), jnp.float32),
            pltpu.VMEM((rpw,), jnp.float32),
            pltpu.VMEM((16,), jnp.float32),
            pltpu.VMEM((128,), jnp.float32),
            pltpu.VMEM((256,), jnp.float32),
            pltpu.SemaphoreType.DMA,
            pltpu.SemaphoreType.DMA,
        ],
        compiler_params=pltpu.CompilerParams(needs_layout_passes=False,
                                             has_side_effects=False),
    )(functools.partial(_sc_body, nchunks, tc_rows))
    return kfn(x, W.reshape(-1), b)


def _mask_kernel(s_ref, k_ref, c_ref, o_ref, sm):
    j = pl.program_id(0)
    s = s_ref[...]
    nz = (s != 0.0)
    mi = nz.astype(jnp.float32)

    @pl.when(j == 0)
    def _init():
        sm[0] = 0

    k = k_ref[0, 0] + jnp.sum(c_ref[...]).astype(jnp.int32)
    start = jnp.maximum(k // 2, 1)
    d = jax.lax.broadcasted_iota(jnp.int32, (128, 128), 0)
    l = jax.lax.broadcasted_iota(jnp.int32, (128, 128), 1)
    tri = (d <= l).astype(jnp.float32)
    incl = jax.lax.dot(mi, tri,
                       preferred_element_type=jnp.float32)
    sel = (d == 127).astype(jnp.float32)
    rowcnt = jax.lax.dot(incl, sel,
                         preferred_element_type=jnp.float32)
    r2 = jax.lax.broadcasted_iota(jnp.int32, (_RB, _RB), 0)
    q2 = jax.lax.broadcasted_iota(jnp.int32, (_RB, _RB), 1)
    low = (q2 < r2).astype(jnp.float32)
    rowoff = jax.lax.dot(low, rowcnt,
                         preferred_element_type=jnp.float32)
    carry = sm[0].astype(jnp.float32)
    rank = carry + rowoff + (incl - mi)
    keep = nz & (rank >= start.astype(jnp.float32))
    o_ref[...] = jnp.where(keep, s, 0.0)
    sm[0] = sm[0] + jnp.sum(mi).astype(jnp.int32)


def kernel(x, W, b):
    N, D = x.shape
    R = N // 128
    tc_rows = _TC_UNITS * _BN
    r_tc = tc_rows // 128
    b2d = b.reshape(1, b.shape[0])

    sums_sc, cnt_sc = _sc_matvec(x, W, b, tc_rows)

    sums_tc, kval = pl.pallas_call(
        _matvec_kernel,
        grid=(tc_rows // _BN,),
        in_specs=[
            pl.BlockSpec((_BN, D), lambda i: (i, 0)),
            pl.BlockSpec((W.shape[0], D), lambda i: (0, 0)),
            pl.BlockSpec((1, b.shape[0]), lambda i: (0, 0)),
        ],
        out_specs=[
            pl.BlockSpec((_BN // 128, 128), lambda i: (i, 0)),
            pl.BlockSpec(memory_space=pltpu.SMEM),
        ],
        out_shape=[
            jax.ShapeDtypeStruct((r_tc, 128), jnp.float32),
            jax.ShapeDtypeStruct((1, 1), jnp.int32),
        ],
        scratch_shapes=[pltpu.SMEM((1,), jnp.int32)],
        compiler_params=pltpu.CompilerParams(
            dimension_semantics=("arbitrary",)),
    )(x, W, b2d)

    sums2d = jnp.concatenate(
        [sums_tc, sums_sc.reshape(-1, 128)], axis=0)
    cnt2d = cnt_sc.reshape(4, 128)

    out2d = pl.pallas_call(
        _mask_kernel,
        grid=(R // _RB,),
        in_specs=[
            pl.BlockSpec((_RB, 128), lambda j: (j, 0)),
            pl.BlockSpec(memory_space=pltpu.SMEM),
            pl.BlockSpec((4, 128), lambda j: (0, 0)),
        ],
        out_specs=pl.BlockSpec((_RB, 128), lambda j: (j, 0)),
        out_shape=jax.ShapeDtypeStruct((R, 128), jnp.float32),
        scratch_shapes=[pltpu.SMEM((1,), jnp.int32)],
        compiler_params=pltpu.CompilerParams(
            dimension_semantics=("arbitrary",)),
    )(sums2d, kval, cnt2d)
    return out2d.reshape(N)


# speedup vs baseline: 8.2183x; 1.4267x over previous
import jax
import jax.numpy as jnp
from jax.experimental import pallas as pl
from jax.experimental.pallas import tpu as pltpu

_BN = 32768
_RB = _BN // 128


def _fused_kernel(xa_ref, xb_ref, w_ref, b_ref, o_ref, s_scr, r_scr, sm):
    p = pl.program_id(0)
    j = pl.program_id(1)

    @pl.when(p == 0)
    def _produce():
        @pl.when(j == 0)
        def _init():
            sm[0] = 0
        wsum = jnp.sum(w_ref[...], axis=0, keepdims=True)
        bsum = jnp.sum(b_ref[...])
        cola = jax.lax.dot_general(
            xa_ref[...], wsum,
            dimension_numbers=(((1,), (1,)), ((), ())),
            preferred_element_type=jnp.float32)
        colb = jax.lax.dot_general(
            xb_ref[...], wsum,
            dimension_numbers=(((1,), (1,)), ((), ())),
            preferred_element_type=jnp.float32)
        s = jnp.concatenate(
            [cola.reshape(_RB // 2, 128), colb.reshape(_RB // 2, 128)],
            axis=0) + bsum
        nz = (s != 0.0)
        mi = nz.astype(jnp.float32)
        d = jax.lax.broadcasted_iota(jnp.int32, (128, 128), 0)
        l = jax.lax.broadcasted_iota(jnp.int32, (128, 128), 1)
        tri = (d <= l).astype(jnp.float32)
        incl = jax.lax.dot(mi, tri,
                           preferred_element_type=jnp.float32)
        sel = (d == 127).astype(jnp.float32)
        rowcnt = jax.lax.dot(incl, sel,
                             preferred_element_type=jnp.float32)
        r2 = jax.lax.broadcasted_iota(jnp.int32, (_RB, _RB), 0)
        q2 = jax.lax.broadcasted_iota(jnp.int32, (_RB, _RB), 1)
        low = (q2 < r2).astype(jnp.float32)
        rowoff = jax.lax.dot(low, rowcnt,
                             preferred_element_type=jnp.float32)
        carry = sm[0].astype(jnp.float32)
        rank = carry + rowoff + (incl - mi)
        s_scr[pl.ds(j * _RB, _RB), :] = s
        r_scr[pl.ds(j * _RB, _RB), :] = rank
        sm[0] = sm[0] + jnp.sum(mi).astype(jnp.int32)

    @pl.when(p == 1)
    def _emit():
        k = sm[0]
        start = jnp.maximum(k // 2, 1).astype(jnp.float32)
        s = s_scr[pl.ds(j * _RB, _RB), :]
        rank = r_scr[pl.ds(j * _RB, _RB), :]
        keep = (s != 0.0) & (rank >= start)
        o_ref[...] = jnp.where(keep, s, 0.0)


def kernel(x, W, b):
    N, D = x.shape
    R = N // 128
    NB = N // _BN
    b2d = b.reshape(1, b.shape[0])
    out2d = pl.pallas_call(
        _fused_kernel,
        grid=(2, NB),
        in_specs=[
            pl.BlockSpec((_BN // 2, D),
                         lambda p, j: (2 * (j * (1 - p) + (NB - 1) * p), 0)),
            pl.BlockSpec((_BN // 2, D),
                         lambda p, j: (2 * (j * (1 - p) + (NB - 1) * p) + 1, 0)),
            pl.BlockSpec((W.shape[0], D), lambda p, j: (0, 0)),
            pl.BlockSpec((1, b.shape[0]), lambda p, j: (0, 0)),
        ],
        out_specs=pl.BlockSpec((_RB, 128), lambda p, j: (j * p, 0)),
        out_shape=jax.ShapeDtypeStruct((R, 128), jnp.float32),
        scratch_shapes=[
            pltpu.VMEM((R, 128), jnp.float32),
            pltpu.VMEM((R, 128), jnp.float32),
            pltpu.SMEM((1,), jnp.int32),
        ],
        compiler_params=pltpu.CompilerParams(
            dimension_semantics=("arbitrary", "arbitrary")),
    )(x, x, W, b2d)
    return out2d.reshape(N)
